# baseline (speedup 1.0000x reference)
import functools

import jax
import jax.numpy as jnp
from jax import lax
from jax.experimental import pallas as pl
from jax.experimental.pallas import tpu as pltpu
from jax.experimental.pallas import tpu_sc as plsc

NC = 2
NS = 16
NW = NC * NS
CHUNK = 128


def kernel(head, relation, tail, timestamp, entity_table, relation_table, timestamp_table):
    B = head.shape[0]
    D = entity_table.shape[1]
    b_per_w = B // NW
    n_chunks = b_per_w // CHUNK

    mesh = plsc.VectorSubcoreMesh(core_axis_name="c", subcore_axis_name="s")

    @functools.partial(
        pl.kernel,
        mesh=mesh,
        compiler_params=pltpu.CompilerParams(use_tc_tiling_on_sc=False),
        out_type=[jax.ShapeDtypeStruct((B, D), jnp.float32)] * 4,
        scratch_types=[
            pltpu.VMEM((n_chunks, CHUNK), jnp.int32),
            pltpu.VMEM((2, CHUNK, D), jnp.float32),
            pltpu.SemaphoreType.DMA,
            pltpu.SemaphoreType.DMA,
        ],
    )
    def gather4(h_i, r_i, t_i, ts_i, ent, rel, tst,
                out_h, out_r, out_t, out_ts,
                idx_v, rows_v, gsem, osem):
        wid = lax.axis_index("s") * NC + lax.axis_index("c")
        tasks = [(h_i, ent, out_h), (r_i, rel, out_r),
                 (t_i, ent, out_t), (ts_i, tst, out_ts)]
        for ix, tab, out in tasks:
            pltpu.sync_copy(ix.at[wid], idx_v)
            for c in range(n_chunks):
                pltpu.async_copy(tab.at[idx_v.at[c]], rows_v.at[c % 2], gsem).wait()
                base = wid * b_per_w + c * CHUNK
                pltpu.sync_copy(rows_v.at[c % 2], out.at[pl.ds(base, CHUNK)])

    idx3 = [x.reshape(NW, n_chunks, CHUNK) for x in (head, relation, tail, timestamp)]
    return tuple(gather4(*idx3, entity_table, relation_table, timestamp_table))


# speedup vs baseline: 1.0127x; 1.0127x over previous
import functools

import jax
import jax.numpy as jnp
from jax import lax
from jax.experimental import pallas as pl
from jax.experimental.pallas import tpu as pltpu
from jax.experimental.pallas import tpu_sc as plsc

NC = 2
NS = 16
NW = NC * NS
CHUNK = 128
K = 4


def kernel(head, relation, tail, timestamp, entity_table, relation_table, timestamp_table):
    B = head.shape[0]
    D = entity_table.shape[1]
    b_per_w = B // NW
    n_chunks = b_per_w // CHUNK
    n_total = 4 * n_chunks
    n_groups = n_total // K

    mesh = plsc.VectorSubcoreMesh(core_axis_name="c", subcore_axis_name="s")

    @functools.partial(
        pl.kernel,
        mesh=mesh,
        compiler_params=pltpu.CompilerParams(use_tc_tiling_on_sc=False),
        out_type=[jax.ShapeDtypeStruct((B, D), jnp.float32)] * 4,
        scratch_types=[
            pltpu.VMEM((4, n_chunks, CHUNK), jnp.int32),
            pltpu.VMEM((2, K, CHUNK, D), jnp.float32),
            pltpu.SemaphoreType.DMA,
            pltpu.SemaphoreType.DMA,
            pltpu.SemaphoreType.DMA,
            pltpu.SemaphoreType.DMA,
        ],
    )
    def gather4(h_i, r_i, t_i, ts_i, ent, rel, tst,
                out_h, out_r, out_t, out_ts,
                idx_v, rows_v, gsem_a, gsem_b, osem_a, osem_b):
        wid = lax.axis_index("s") * NC + lax.axis_index("c")
        tables = [ent, rel, ent, tst]
        outs = [out_h, out_r, out_t, out_ts]
        for task, ix in enumerate([h_i, r_i, t_i, ts_i]):
            pltpu.sync_copy(ix.at[wid], idx_v.at[task])

        gsems = [gsem_a, gsem_b]
        osems = [osem_a, osem_b]

        def fire_gathers(g, s):
            descs = []
            for j in range(K):
                q = g * K + j
                task, c = q // n_chunks, q % n_chunks
                descs.append(pltpu.async_copy(
                    tables[task].at[idx_v.at[task, c]], rows_v.at[s, j], gsems[s]))
            return descs

        def fire_outcopies(g, s):
            descs = []
            for j in range(K):
                q = g * K + j
                task, c = q // n_chunks, q % n_chunks
                base = wid * b_per_w + c * CHUNK
                descs.append(pltpu.async_copy(
                    rows_v.at[s, j], outs[task].at[pl.ds(base, CHUNK)], osems[s]))
            return descs

        gd = {0: fire_gathers(0, 0)}
        od = {}
        for g in range(n_groups):
            s = g % 2
            if g + 1 < n_groups:
                if g >= 1:
                    for d in od.pop(g - 1):
                        d.wait()
                gd[g + 1] = fire_gathers(g + 1, 1 - s)
            for d in gd.pop(g):
                d.wait()
            od[g] = fire_outcopies(g, s)
        for g in sorted(od):
            for d in od.pop(g):
                d.wait()

    idx3 = [x.reshape(NW, n_chunks, CHUNK) for x in (head, relation, tail, timestamp)]
    return tuple(gather4(*idx3, entity_table, relation_table, timestamp_table))
